# Initial kernel scaffold; baseline (speedup 1.0000x reference)
import jax
import jax.numpy as jnp
from jax.experimental import pallas as pl


def kernel(x, edge_index, size, w, bias):
    raise NotImplementedError("write your pallas kernel here")


# speedup vs baseline: 7.9089x; 7.9089x over previous
import functools

import jax
import jax.numpy as jnp
from jax import lax
from jax.experimental import pallas as pl
from jax.experimental.pallas import tpu as pltpu
from jax.experimental.pallas import tpu_sc as plsc

NC = 2
NS = 16
DP = 144


def _sc_aggregate(xp, row, col, zblk, *, n, e):
    nw = NC * NS
    ept = e // nw
    ch = 40
    nch = ept // ch
    rpt = n // NS
    rb = zblk.shape[0]
    nrb = rpt // rb

    mesh = plsc.VectorSubcoreMesh(
        core_axis_name="c", subcore_axis_name="s", num_cores=NC, num_subcores=NS
    )

    @functools.partial(
        pl.kernel,
        out_type=jax.ShapeDtypeStruct((NC, n, DP), jnp.float32),
        mesh=mesh,
        compiler_params=pltpu.CompilerParams(use_tc_tiling_on_sc=False),
        scratch_types=[
            pltpu.VMEM_SHARED((n, DP), jnp.float32),
            pltpu.VMEM((rb, DP), jnp.float32),
            pltpu.VMEM((ch,), jnp.int32),
            pltpu.VMEM((ch,), jnp.int32),
            pltpu.VMEM((ch, DP), jnp.float32),
            pltpu.SemaphoreType.DMA,
        ],
    )
    def k(xp_hbm, row_hbm, col_hbm, z_hbm, out_hbm, acc, zbuf, colbuf, rowbuf, rows, gsem):
        c = lax.axis_index("c")
        s = lax.axis_index("s")
        pltpu.sync_copy(z_hbm, zbuf)
        for r in range(nrb):
            pltpu.sync_copy(zbuf, acc.at[pl.ds(s * rpt + r * rb, rb)])
        plsc.subcore_barrier()

        ebase = (c * NS + s) * ept

        def body(kk, carry):
            off = ebase + kk * ch
            pltpu.sync_copy(col_hbm.at[pl.ds(off, ch)], colbuf)
            pltpu.async_copy(xp_hbm.at[colbuf], rows, gsem).wait()
            pltpu.sync_copy(row_hbm.at[pl.ds(off, ch)], rowbuf)
            pltpu.sync_copy(rows, acc.at[rowbuf], add=True)
            return carry

        lax.fori_loop(0, nch, body, 0)
        plsc.subcore_barrier()
        for r in range(nrb):
            base = s * rpt + r * rb
            pltpu.sync_copy(acc.at[pl.ds(base, rb)], zbuf)
            pltpu.sync_copy(zbuf, out_hbm.at[c, pl.ds(base, rb)])

    return k(xp, row, col, zblk)


def _tc_finish(partials, w, bias2, *, n, d_out):
    blk = 1000

    def body(p0_ref, p1_ref, w_ref, b_ref, o_ref):
        ssum = p0_ref[0] + p1_ref[0]
        feat = ssum[:, :d_out]
        deg = ssum[:, d_out:d_out + 1]
        dinv = jnp.where(deg > 0, lax.rsqrt(deg), 0.0)
        o_ref[...] = (
            jnp.dot(feat * dinv, w_ref[...], preferred_element_type=jnp.float32)
            + b_ref[...]
        )

    return pl.pallas_call(
        body,
        grid=(n // blk,),
        in_specs=[
            pl.BlockSpec((1, blk, DP), lambda i: (0, i, 0)),
            pl.BlockSpec((1, blk, DP), lambda i: (1, i, 0)),
            pl.BlockSpec(w.shape, lambda i: (0, 0)),
            pl.BlockSpec((1, d_out), lambda i: (0, 0)),
        ],
        out_specs=pl.BlockSpec((blk, d_out), lambda i: (i, 0)),
        out_shape=jax.ShapeDtypeStruct((n, d_out), jnp.float32),
    )(partials, partials, w, bias2)


def kernel(x, edge_index, size, w, bias):
    n, d = x.shape
    e = edge_index.shape[1]
    d_out = w.shape[1]
    row = edge_index[0]
    col = edge_index[1]
    xp = jnp.concatenate(
        [x, jnp.ones((n, 1), x.dtype), jnp.zeros((n, DP - d - 1), x.dtype)], axis=1
    )
    zblk = jnp.zeros((125, DP), jnp.float32)
    partials = _sc_aggregate(xp, row, col, zblk, n=n, e=e)
    shift = (jnp.asarray(size) - n).astype(x.dtype)
    bias2 = (bias + shift).reshape(1, d_out)
    return _tc_finish(partials, w, bias2, n=n, d_out=d_out)


# speedup vs baseline: 21.8064x; 2.7572x over previous
import functools

import jax
import jax.numpy as jnp
from jax import lax
from jax.experimental import pallas as pl
from jax.experimental.pallas import tpu as pltpu
from jax.experimental.pallas import tpu_sc as plsc

NC = 2
NS = 16
DP = 144


def _sc_aggregate(xp, row, col, zblk, *, n, e):
    nw = NC * NS
    ept = e // nw
    ch = 80
    nch = ept // ch
    nbuf = 2
    rpt = n // NS
    rb = zblk.shape[0]
    nrb = rpt // rb

    mesh = plsc.VectorSubcoreMesh(
        core_axis_name="c", subcore_axis_name="s", num_cores=NC, num_subcores=NS
    )

    @functools.partial(
        pl.kernel,
        out_type=jax.ShapeDtypeStruct((NC, n, DP), jnp.float32),
        mesh=mesh,
        compiler_params=pltpu.CompilerParams(use_tc_tiling_on_sc=False),
        scratch_types=[
            pltpu.VMEM_SHARED((n, DP), jnp.float32),
            pltpu.VMEM((ept,), jnp.int32),
            [pltpu.VMEM((ch, DP), jnp.float32) for _ in range(nbuf)],
            [pltpu.VMEM((ch,), jnp.int32) for _ in range(nbuf)],
            [pltpu.SemaphoreType.DMA for _ in range(nbuf)],
            [pltpu.SemaphoreType.DMA for _ in range(nbuf)],
        ],
    )
    def k(xp_hbm, row_hbm, col_hbm, z_hbm, out_hbm, acc, colbuf, rows, ridx, gsem, isem):
        c = lax.axis_index("c")
        s = lax.axis_index("s")
        wid = c * NS + s
        ebase = wid * ept
        pltpu.sync_copy(col_hbm.at[pl.ds(ebase, ept)], colbuf)
        for r in range(nrb):
            pltpu.sync_copy(z_hbm, acc.at[pl.ds(s * rpt + r * rb, rb)])
        plsc.subcore_barrier()

        def fetch(kk, b):
            pltpu.async_copy(row_hbm.at[pl.ds(ebase + kk * ch, ch)], ridx[b], isem[b])
            pltpu.async_copy(xp_hbm.at[colbuf.at[pl.ds(kk * ch, ch)]], rows[b], gsem[b])

        def drain(b):
            pltpu.make_async_copy(row_hbm.at[pl.ds(0, ch)], ridx[b], isem[b]).wait()
            pltpu.make_async_copy(xp_hbm.at[pl.ds(0, ch)], rows[b], gsem[b]).wait()

        def scatter(b):
            pltpu.sync_copy(rows[b], acc.at[ridx[b]], add=True)

        for b in range(nbuf):
            fetch(b, b)

        def body(i, carry):
            for b in range(nbuf):
                kk = i * nbuf + b
                drain(b)
                scatter(b)

                @pl.when(kk + nbuf < nch)
                def _():
                    fetch(kk + nbuf, b)

            return carry

        lax.fori_loop(0, nch // nbuf, body, 0)
        for kk in range(nch - nch % nbuf, nch):
            b = kk % nbuf
            drain(b)
            scatter(b)
        plsc.subcore_barrier()
        for r in range(nrb):
            base = s * rpt + r * rb
            pltpu.sync_copy(acc.at[pl.ds(base, rb)], out_hbm.at[c, pl.ds(base, rb)])

    return k(xp, row, col, zblk)


def _tc_finish(partials, w, bias2, *, n, d_out):
    blk = 1000

    def body(p0_ref, p1_ref, w_ref, b_ref, o_ref):
        ssum = p0_ref[0] + p1_ref[0]
        feat = ssum[:, :d_out]
        deg = ssum[:, d_out:d_out + 1]
        dinv = jnp.where(deg > 0, lax.rsqrt(deg), 0.0)
        o_ref[...] = (
            jnp.dot(feat * dinv, w_ref[...], preferred_element_type=jnp.float32)
            + b_ref[...]
        )

    return pl.pallas_call(
        body,
        grid=(n // blk,),
        in_specs=[
            pl.BlockSpec((1, blk, DP), lambda i: (0, i, 0)),
            pl.BlockSpec((1, blk, DP), lambda i: (1, i, 0)),
            pl.BlockSpec(w.shape, lambda i: (0, 0)),
            pl.BlockSpec((1, d_out), lambda i: (0, 0)),
        ],
        out_specs=pl.BlockSpec((blk, d_out), lambda i: (i, 0)),
        out_shape=jax.ShapeDtypeStruct((n, d_out), jnp.float32),
    )(partials, partials, w, bias2)


def kernel(x, edge_index, size, w, bias):
    n, d = x.shape
    e = edge_index.shape[1]
    d_out = w.shape[1]
    row = edge_index[0]
    col = edge_index[1]
    xp = jnp.concatenate(
        [x, jnp.ones((n, 1), x.dtype), jnp.zeros((n, DP - d - 1), x.dtype)], axis=1
    )
    zblk = jnp.zeros((125, DP), jnp.float32)
    partials = _sc_aggregate(xp, row, col, zblk, n=n, e=e)
    shift = (jnp.asarray(size) - n).astype(x.dtype)
    bias2 = (bias + shift).reshape(1, d_out)
    return _tc_finish(partials, w, bias2, n=n, d_out=d_out)


# speedup vs baseline: 22.6884x; 1.0404x over previous
import functools

import jax
import jax.numpy as jnp
from jax import lax
from jax.experimental import pallas as pl
from jax.experimental.pallas import tpu as pltpu
from jax.experimental.pallas import tpu_sc as plsc

NC = 2
NS = 16
DP = 144


def _sc_aggregate(xp, edge_index, zblk, *, n, e):
    nw = NC * NS
    ept = e // nw
    ch = 40
    nch = ept // ch
    nbuf = 4
    rpt = n // NS
    rb = zblk.shape[0]
    nrb = rpt // rb

    mesh = plsc.VectorSubcoreMesh(
        core_axis_name="c", subcore_axis_name="s", num_cores=NC, num_subcores=NS
    )

    @functools.partial(
        pl.kernel,
        out_type=jax.ShapeDtypeStruct((NC, n, DP), jnp.float32),
        mesh=mesh,
        compiler_params=pltpu.CompilerParams(use_tc_tiling_on_sc=False),
        scratch_types=[
            pltpu.VMEM_SHARED((n, DP), jnp.float32),
            pltpu.VMEM((ept,), jnp.int32),
            [pltpu.VMEM((ch, DP), jnp.float32) for _ in range(nbuf)],
            [pltpu.VMEM((ch,), jnp.int32) for _ in range(nbuf)],
            [pltpu.SemaphoreType.DMA for _ in range(nbuf)],
            [pltpu.SemaphoreType.DMA for _ in range(nbuf)],
            [pltpu.SemaphoreType.DMA for _ in range(nbuf)],
        ],
    )
    def k(xp_hbm, ei_hbm, z_hbm, out_hbm, acc, colbuf, rows, ridx, gsem, isem, ssem):
        c = lax.axis_index("c")
        s = lax.axis_index("s")
        wid = c * NS + s
        ebase = wid * ept
        pltpu.sync_copy(ei_hbm.at[1, pl.ds(ebase, ept)], colbuf)
        for r in range(nrb):
            pltpu.sync_copy(z_hbm, acc.at[pl.ds(s * rpt + r * rb, rb)])
        plsc.subcore_barrier()

        def fetch(kk, b):
            pltpu.async_copy(ei_hbm.at[0, pl.ds(ebase + kk * ch, ch)], ridx[b], isem[b])
            pltpu.async_copy(xp_hbm.at[colbuf.at[pl.ds(kk * ch, ch)]], rows[b], gsem[b])

        def drain_fetch(b):
            pltpu.make_async_copy(ei_hbm.at[0, pl.ds(0, ch)], ridx[b], isem[b]).wait()
            pltpu.make_async_copy(xp_hbm.at[pl.ds(0, ch)], rows[b], gsem[b]).wait()

        for b in range(nbuf):
            fetch(b, b)

        def body(i, carry):
            descs = []
            for b in range(nbuf):
                drain_fetch(b)
                descs.append(
                    pltpu.async_copy(rows[b], acc.at[ridx[b]], ssem[b], add=True)
                )
            for b in range(nbuf):
                kk = (i + 1) * nbuf + b
                descs[b].wait()

                @pl.when(kk < nch)
                def _():
                    fetch(kk, b)

            return carry

        lax.fori_loop(0, nch // nbuf, body, 0)
        for kk in range(nch - nch % nbuf, nch):
            b = kk % nbuf
            drain_fetch(b)
            pltpu.sync_copy(rows[b], acc.at[ridx[b]], add=True)
        plsc.subcore_barrier()
        for r in range(nrb):
            base = s * rpt + r * rb
            pltpu.sync_copy(acc.at[pl.ds(base, rb)], out_hbm.at[c, pl.ds(base, rb)])

    return k(xp, edge_index, zblk)


def _tc_finish(partials, w, bias2, *, n, d_out):
    blk = 1000

    def body(p0_ref, p1_ref, w_ref, b_ref, o_ref):
        ssum = p0_ref[0] + p1_ref[0]
        feat = ssum[:, :d_out]
        deg = ssum[:, d_out:d_out + 1]
        dinv = jnp.where(deg > 0, lax.rsqrt(deg), 0.0)
        o_ref[...] = (
            jnp.dot(feat * dinv, w_ref[...], preferred_element_type=jnp.float32)
            + b_ref[...]
        )

    return pl.pallas_call(
        body,
        grid=(n // blk,),
        in_specs=[
            pl.BlockSpec((1, blk, DP), lambda i: (0, i, 0)),
            pl.BlockSpec((1, blk, DP), lambda i: (1, i, 0)),
            pl.BlockSpec(w.shape, lambda i: (0, 0)),
            pl.BlockSpec((1, d_out), lambda i: (0, 0)),
        ],
        out_specs=pl.BlockSpec((blk, d_out), lambda i: (i, 0)),
        out_shape=jax.ShapeDtypeStruct((n, d_out), jnp.float32),
    )(partials, partials, w, bias2)


def kernel(x, edge_index, size, w, bias):
    n, d = x.shape
    e = edge_index.shape[1]
    d_out = w.shape[1]
    xp = jnp.concatenate(
        [x, jnp.ones((n, 1), x.dtype), jnp.zeros((n, DP - d - 1), x.dtype)], axis=1
    )
    zblk = jnp.zeros((125, DP), jnp.float32)
    partials = _sc_aggregate(xp, edge_index, zblk, n=n, e=e)
    shift = (jnp.asarray(size) - n).astype(x.dtype)
    bias2 = (bias + shift).reshape(1, d_out)
    return _tc_finish(partials, w, bias2, n=n, d_out=d_out)
